# baseline (speedup 1.0000x reference)
import jax
import jax.numpy as jnp
from jax.experimental import pallas as pl

_X_DIM = 128
_Y_DIM = 16
_N_BINS = 128
_HIDDEN = 64
_TILE = 2048


def _loss_kernel(x_ref, y_ref, w1_ref, b1_ref, w2_ref, b2_ref, out_ref):
    x = x_ref[...]
    y = y_ref[...]
    h = jnp.maximum(
        jax.lax.dot_general(
            x, w1_ref[...], (((1,), (0,)), ((), ())),
            preferred_element_type=jnp.float32,
        ) + b1_ref[...],
        0.0,
    )
    logits = jax.lax.dot_general(
        h, w2_ref[...], (((1,), (0,)), ((), ())),
        preferred_element_type=jnp.float32,
    ) + b2_ref[...]

    e = jnp.exp2(logits)
    r = jax.lax.broadcasted_iota(jnp.int32, (_Y_DIM * _N_BINS, _Y_DIM), 0)
    c = jax.lax.broadcasted_iota(jnp.int32, (_Y_DIM * _N_BINS, _Y_DIM), 1)
    g = (r // _N_BINS == c).astype(jnp.float32)
    sums = jax.lax.dot_general(
        e, g, (((1,), (0,)), ((), ())),
        preferred_element_type=jnp.float32,
    )
    lse_sum = jnp.sum(jnp.log2(sums))

    y_clipped = jnp.clip(y, -0.99, 0.99)
    y_true_cont = (y_clipped + 1.0) / 2.0 * _N_BINS
    labels = jnp.floor(y_true_cont).astype(jnp.int32)

    tile = x.shape[0]
    col = jax.lax.broadcasted_iota(jnp.int32, (tile, _N_BINS), 1)
    pick_acc = jnp.zeros((tile, _N_BINS), dtype=jnp.float32)
    for i in range(_Y_DIM):
        s = logits[:, i * _N_BINS:(i + 1) * _N_BINS]
        pick_acc = pick_acc + jnp.where(col == labels[:, i:i + 1], s, 0.0)
    picked_sum = jnp.sum(pick_acc)

    tile_sum = (lse_sum - picked_sum).reshape(1, 1)

    @pl.when(pl.program_id(0) == 0)
    def _init():
        out_ref[...] = jnp.zeros_like(out_ref)

    out_ref[...] += tile_sum

    @pl.when(pl.program_id(0) == pl.num_programs(0) - 1)
    def _finish():
        out_ref[...] = out_ref[...] * (0.6931471805599453 / (pl.num_programs(0) * tile))


def kernel(x_batch, y_batch, W1, b1, W2, b2):
    n = x_batch.shape[0]
    log2e = 1.4426950408889634
    w1_x = W1[_Y_DIM:_Y_DIM + _X_DIM, :]
    b1r = b1.reshape(1, _HIDDEN)
    w2_s = W2 * log2e
    b2r = (b2 * log2e).reshape(1, _Y_DIM * _N_BINS)

    grid = (n // _TILE,)
    total = pl.pallas_call(
        _loss_kernel,
        grid=grid,
        in_specs=[
            pl.BlockSpec((_TILE, _X_DIM), lambda i: (i, 0)),
            pl.BlockSpec((_TILE, _Y_DIM), lambda i: (i, 0)),
            pl.BlockSpec((_X_DIM, _HIDDEN), lambda i: (0, 0)),
            pl.BlockSpec((1, _HIDDEN), lambda i: (0, 0)),
            pl.BlockSpec((_HIDDEN, _Y_DIM * _N_BINS), lambda i: (0, 0)),
            pl.BlockSpec((1, _Y_DIM * _N_BINS), lambda i: (0, 0)),
        ],
        out_specs=pl.BlockSpec((1, 1), lambda i: (0, 0)),
        out_shape=jax.ShapeDtypeStruct((1, 1), jnp.float32),
    )(x_batch, y_batch, w1_x, b1r, w2_s, b2r)
    return total.reshape(())


# speedup vs baseline: 1.0577x; 1.0577x over previous
import jax
import jax.numpy as jnp
from jax.experimental import pallas as pl

_X_DIM = 128
_Y_DIM = 16
_N_BINS = 128
_HIDDEN = 64
_TILE = 2048


def _loss_kernel(x_ref, y_ref, w1_ref, b1_ref, w2_ref, b2_ref, out_ref):
    x = x_ref[...]
    y = y_ref[...]
    h = jnp.maximum(
        jax.lax.dot_general(
            x, w1_ref[...], (((1,), (0,)), ((), ())),
            preferred_element_type=jnp.float32,
        ) + b1_ref[...],
        0.0,
    ) * 1.4426950408889634
    logits = jax.lax.dot_general(
        h, w2_ref[...], (((1,), (0,)), ((), ())),
        preferred_element_type=jnp.float32,
    ) + b2_ref[...] * 1.4426950408889634

    e = jnp.exp2(logits)
    r = jax.lax.broadcasted_iota(jnp.int32, (_Y_DIM * _N_BINS, _Y_DIM), 0)
    c = jax.lax.broadcasted_iota(jnp.int32, (_Y_DIM * _N_BINS, _Y_DIM), 1)
    g = (r // _N_BINS == c).astype(jnp.float32)
    sums = jax.lax.dot_general(
        e, g, (((1,), (0,)), ((), ())),
        preferred_element_type=jnp.float32,
    )
    lse_sum = jnp.sum(jnp.log2(sums))

    y_clipped = jnp.clip(y, -0.99, 0.99)
    y_true_cont = (y_clipped + 1.0) / 2.0 * _N_BINS
    labels = jnp.floor(y_true_cont).astype(jnp.int32)

    tile = x.shape[0]
    col = jax.lax.broadcasted_iota(jnp.int32, (tile, _N_BINS), 1)
    pick_acc = jnp.zeros((tile, _N_BINS), dtype=jnp.float32)
    for i in range(_Y_DIM):
        s = logits[:, i * _N_BINS:(i + 1) * _N_BINS]
        pick_acc = pick_acc + jnp.where(col == labels[:, i:i + 1], s, 0.0)
    picked_sum = jnp.sum(pick_acc)

    tile_sum = (lse_sum - picked_sum).reshape(1, 1)

    @pl.when(pl.program_id(0) == 0)
    def _init():
        out_ref[...] = jnp.zeros_like(out_ref)

    out_ref[...] += tile_sum

    @pl.when(pl.program_id(0) == pl.num_programs(0) - 1)
    def _finish():
        out_ref[...] = out_ref[...] * (0.6931471805599453 / (pl.num_programs(0) * tile))


def kernel(x_batch, y_batch, W1, b1, W2, b2):
    n = x_batch.shape[0]
    w1_x = W1[_Y_DIM:_Y_DIM + _X_DIM, :]
    b1r = b1.reshape(1, _HIDDEN)
    b2r = b2.reshape(1, _Y_DIM * _N_BINS)

    grid = (n // _TILE,)
    total = pl.pallas_call(
        _loss_kernel,
        grid=grid,
        in_specs=[
            pl.BlockSpec((_TILE, _X_DIM), lambda i: (i, 0)),
            pl.BlockSpec((_TILE, _Y_DIM), lambda i: (i, 0)),
            pl.BlockSpec((_X_DIM, _HIDDEN), lambda i: (0, 0)),
            pl.BlockSpec((1, _HIDDEN), lambda i: (0, 0)),
            pl.BlockSpec((_HIDDEN, _Y_DIM * _N_BINS), lambda i: (0, 0)),
            pl.BlockSpec((1, _Y_DIM * _N_BINS), lambda i: (0, 0)),
        ],
        out_specs=pl.BlockSpec((1, 1), lambda i: (0, 0)),
        out_shape=jax.ShapeDtypeStruct((1, 1), jnp.float32),
    )(x_batch, y_batch, w1_x, b1r, W2, b2r)
    return total.reshape(())


# speedup vs baseline: 1.0621x; 1.0041x over previous
import jax
import jax.numpy as jnp
from jax.experimental import pallas as pl

_X_DIM = 128
_Y_DIM = 16
_N_BINS = 128
_HIDDEN = 64
_TILE = 2048


def _loss_kernel(x_ref, y_ref, w1_ref, b1_ref, w2_ref, b2_ref, out_ref):
    x = x_ref[...]
    y = y_ref[...]
    in_dim = w1_ref.shape[0]
    pr = jax.lax.broadcasted_iota(jnp.int32, (_X_DIM, in_dim), 0)
    pc = jax.lax.broadcasted_iota(jnp.int32, (_X_DIM, in_dim), 1)
    psel = (pc == pr + _Y_DIM).astype(jnp.float32)
    w1x = jax.lax.dot_general(
        psel, w1_ref[...], (((1,), (0,)), ((), ())),
        preferred_element_type=jnp.float32,
    )
    h = jnp.maximum(
        jax.lax.dot_general(
            x, w1x, (((1,), (0,)), ((), ())),
            preferred_element_type=jnp.float32,
        ) + b1_ref[...],
        0.0,
    ) * 1.4426950408889634
    logits = jax.lax.dot_general(
        h, w2_ref[...], (((1,), (0,)), ((), ())),
        preferred_element_type=jnp.float32,
    ) + b2_ref[...] * 1.4426950408889634

    e = jnp.exp2(logits)
    r = jax.lax.broadcasted_iota(jnp.int32, (_Y_DIM * _N_BINS, _Y_DIM), 0)
    c = jax.lax.broadcasted_iota(jnp.int32, (_Y_DIM * _N_BINS, _Y_DIM), 1)
    g = (r // _N_BINS == c).astype(jnp.float32)
    sums = jax.lax.dot_general(
        e, g, (((1,), (0,)), ((), ())),
        preferred_element_type=jnp.float32,
    )
    lse_sum = jnp.sum(jnp.log2(sums))

    y_clipped = jnp.clip(y, -0.99, 0.99)
    y_true_cont = (y_clipped + 1.0) / 2.0 * _N_BINS
    labels = jnp.floor(y_true_cont).astype(jnp.int32)

    tile = x.shape[0]
    col = jax.lax.broadcasted_iota(jnp.int32, (tile, _N_BINS), 1)
    pick_acc = jnp.zeros((tile, _N_BINS), dtype=jnp.float32)
    for i in range(_Y_DIM):
        s = logits[:, i * _N_BINS:(i + 1) * _N_BINS]
        pick_acc = pick_acc + jnp.where(col == labels[:, i:i + 1], s, 0.0)
    picked_sum = jnp.sum(pick_acc)

    tile_sum = (lse_sum - picked_sum).reshape(1, 1)

    @pl.when(pl.program_id(0) == 0)
    def _init():
        out_ref[...] = jnp.zeros_like(out_ref)

    out_ref[...] += tile_sum

    @pl.when(pl.program_id(0) == pl.num_programs(0) - 1)
    def _finish():
        out_ref[...] = out_ref[...] * (0.6931471805599453 / (pl.num_programs(0) * tile))


def kernel(x_batch, y_batch, W1, b1, W2, b2):
    n = x_batch.shape[0]
    in_dim = W1.shape[0]
    b1r = b1.reshape(1, _HIDDEN)
    b2r = b2.reshape(1, _Y_DIM * _N_BINS)

    grid = (n // _TILE,)
    total = pl.pallas_call(
        _loss_kernel,
        grid=grid,
        in_specs=[
            pl.BlockSpec((_TILE, _X_DIM), lambda i: (i, 0)),
            pl.BlockSpec((_TILE, _Y_DIM), lambda i: (i, 0)),
            pl.BlockSpec((in_dim, _HIDDEN), lambda i: (0, 0)),
            pl.BlockSpec((1, _HIDDEN), lambda i: (0, 0)),
            pl.BlockSpec((_HIDDEN, _Y_DIM * _N_BINS), lambda i: (0, 0)),
            pl.BlockSpec((1, _Y_DIM * _N_BINS), lambda i: (0, 0)),
        ],
        out_specs=pl.BlockSpec((1, 1), lambda i: (0, 0)),
        out_shape=jax.ShapeDtypeStruct((1, 1), jnp.float32),
    )(x_batch, y_batch, W1, b1r, W2, b2r)
    return total.reshape(())


# speedup vs baseline: 1.0713x; 1.0087x over previous
import jax
import jax.numpy as jnp
from jax.experimental import pallas as pl

_X_DIM = 128
_Y_DIM = 16
_N_BINS = 128
_HIDDEN = 64
_TILE = 4096


def _loss_kernel(x_ref, y_ref, w1_ref, b1_ref, w2_ref, b2_ref, out_ref):
    x = x_ref[...]
    y = y_ref[...]
    in_dim = w1_ref.shape[0]
    pr = jax.lax.broadcasted_iota(jnp.int32, (_X_DIM, in_dim), 0)
    pc = jax.lax.broadcasted_iota(jnp.int32, (_X_DIM, in_dim), 1)
    psel = (pc == pr + _Y_DIM).astype(jnp.float32)
    w1x = jax.lax.dot_general(
        psel, w1_ref[...], (((1,), (0,)), ((), ())),
        preferred_element_type=jnp.float32,
    )
    h = jnp.maximum(
        jax.lax.dot_general(
            x, w1x, (((1,), (0,)), ((), ())),
            preferred_element_type=jnp.float32,
        ) + b1_ref[...],
        0.0,
    ) * 1.4426950408889634
    logits = jax.lax.dot_general(
        h, w2_ref[...], (((1,), (0,)), ((), ())),
        preferred_element_type=jnp.float32,
    ) + b2_ref[...] * 1.4426950408889634

    e = jnp.exp2(logits)
    r = jax.lax.broadcasted_iota(jnp.int32, (_Y_DIM * _N_BINS, _Y_DIM), 0)
    c = jax.lax.broadcasted_iota(jnp.int32, (_Y_DIM * _N_BINS, _Y_DIM), 1)
    g = (r // _N_BINS == c).astype(jnp.float32)
    sums = jax.lax.dot_general(
        e, g, (((1,), (0,)), ((), ())),
        preferred_element_type=jnp.float32,
    )
    lse_sum = jnp.sum(jnp.log2(sums))

    y_clipped = jnp.clip(y, -0.99, 0.99)
    y_true_cont = (y_clipped + 1.0) / 2.0 * _N_BINS
    labels = jnp.floor(y_true_cont).astype(jnp.int32)

    tile = x.shape[0]
    col = jax.lax.broadcasted_iota(jnp.int32, (tile, _N_BINS), 1)
    pick_acc = jnp.zeros((tile, _N_BINS), dtype=jnp.float32)
    for i in range(_Y_DIM):
        s = logits[:, i * _N_BINS:(i + 1) * _N_BINS]
        pick_acc = pick_acc + jnp.where(col == labels[:, i:i + 1], s, 0.0)
    picked_sum = jnp.sum(pick_acc)

    tile_sum = (lse_sum - picked_sum).reshape(1, 1)

    @pl.when(pl.program_id(0) == 0)
    def _init():
        out_ref[...] = jnp.zeros_like(out_ref)

    out_ref[...] += tile_sum

    @pl.when(pl.program_id(0) == pl.num_programs(0) - 1)
    def _finish():
        out_ref[...] = out_ref[...] * (0.6931471805599453 / (pl.num_programs(0) * tile))


def kernel(x_batch, y_batch, W1, b1, W2, b2):
    n = x_batch.shape[0]
    in_dim = W1.shape[0]
    b1r = b1.reshape(1, _HIDDEN)
    b2r = b2.reshape(1, _Y_DIM * _N_BINS)

    grid = (n // _TILE,)
    total = pl.pallas_call(
        _loss_kernel,
        grid=grid,
        in_specs=[
            pl.BlockSpec((_TILE, _X_DIM), lambda i: (i, 0)),
            pl.BlockSpec((_TILE, _Y_DIM), lambda i: (i, 0)),
            pl.BlockSpec((in_dim, _HIDDEN), lambda i: (0, 0)),
            pl.BlockSpec((1, _HIDDEN), lambda i: (0, 0)),
            pl.BlockSpec((_HIDDEN, _Y_DIM * _N_BINS), lambda i: (0, 0)),
            pl.BlockSpec((1, _Y_DIM * _N_BINS), lambda i: (0, 0)),
        ],
        out_specs=pl.BlockSpec((1, 1), lambda i: (0, 0)),
        out_shape=jax.ShapeDtypeStruct((1, 1), jnp.float32),
    )(x_batch, y_batch, W1, b1r, W2, b2r)
    return total.reshape(())
